# baseline (speedup 1.0000x reference)
import jax
import jax.numpy as jnp
from jax.experimental import pallas as pl

_B = 16384
_D = 32
_ROWS_BLK = 4096
_ROWS_BLK2 = 16384


def _stage1_body(ut_ref, dt_ref, w1u_ref, w1d_ref, b1_ref, p_ref):
    p = jnp.dot(w1u_ref[...], ut_ref[...],
                preferred_element_type=jnp.float32)
    p = p + jnp.dot(w1d_ref[...], dt_ref[...],
                    preferred_element_type=jnp.float32)
    p_ref[...] = p + b1_ref[...]


def _stage2_body(ut_ref, mt_ref, p_ref, w1e_ref, w1m_ref,
                 w2_ref, b2_ref, o_ref):
    ut = ut_ref[...]
    mt = mt_ref[...]
    acc = p_ref[...]
    acc = acc + jnp.dot(w1e_ref[...], ut * mt,
                        preferred_element_type=jnp.float32)
    acc = acc + jnp.dot(w1m_ref[...], mt,
                        preferred_element_type=jnp.float32)
    h = jnp.maximum(acc, 0.0)
    z = jnp.sum(h * w2_ref[...], axis=0, keepdims=True) + b2_ref[...]
    o_ref[...] = 1.0 / (1.0 + jnp.exp(-z))


def kernel(data, user_table, movie_table, W1, b1, W2, b2):
    uid = data[:, 0].astype(jnp.int32)
    mid = data[:, 1].astype(jnp.int32)

    u_emb = user_table.at[uid].get(mode="promise_in_bounds")
    m_emb = movie_table.at[mid].get(mode="promise_in_bounds")

    w1e = W1[0:_D].T
    w1u = W1[_D:2 * _D].T
    w1m = W1[2 * _D:3 * _D].T
    w1d = jnp.concatenate(
        [jnp.zeros((2, W1.shape[1]), W1.dtype), W1[3 * _D:]], axis=0).T
    b1c = b1.reshape(-1, 1)
    w2c = W2.reshape(-1, 1)
    b2c = b2.reshape(1, 1)

    nblk = _B // _ROWS_BLK
    partial = pl.pallas_call(
        _stage1_body,
        grid=(nblk,),
        in_specs=[
            pl.BlockSpec((_D, _ROWS_BLK), lambda i: (0, i)),
            pl.BlockSpec((44, _ROWS_BLK), lambda i: (0, i)),
            pl.BlockSpec((24, _D), lambda i: (0, 0)),
            pl.BlockSpec((24, 44), lambda i: (0, 0)),
            pl.BlockSpec((24, 1), lambda i: (0, 0)),
        ],
        out_specs=pl.BlockSpec((24, _ROWS_BLK), lambda i: (0, i)),
        out_shape=jax.ShapeDtypeStruct((24, _B), jnp.float32),
    )(u_emb.T, data.T, w1u, w1d, b1c)

    out = pl.pallas_call(
        _stage2_body,
        grid=(_B // _ROWS_BLK2,),
        in_specs=[
            pl.BlockSpec((_D, _ROWS_BLK2), lambda i: (0, i)),
            pl.BlockSpec((_D, _ROWS_BLK2), lambda i: (0, i)),
            pl.BlockSpec((24, _ROWS_BLK2), lambda i: (0, i)),
            pl.BlockSpec((24, _D), lambda i: (0, 0)),
            pl.BlockSpec((24, _D), lambda i: (0, 0)),
            pl.BlockSpec((24, 1), lambda i: (0, 0)),
            pl.BlockSpec((1, 1), lambda i: (0, 0)),
        ],
        out_specs=pl.BlockSpec((1, _ROWS_BLK2), lambda i: (0, i)),
        out_shape=jax.ShapeDtypeStruct((1, _B), jnp.float32),
    )(u_emb.T, m_emb.T, partial, w1e, w1m, w2c, b2c)
    return out.reshape(_B, 1)
